# baseline (speedup 1.0000x reference)
import functools

import jax
import jax.numpy as jnp
from jax import lax
from jax.experimental import pallas as pl
from jax.experimental.pallas import tpu as pltpu
from jax.experimental.pallas import tpu_sc as plsc

BATCH = 16384
DIM = 32
NUM_CORES = 2
NUM_SUBCORES = 16
LANES = 16
NUM_WORKERS = NUM_CORES * NUM_SUBCORES
BPW = BATCH // NUM_WORKERS
GROUPS = BPW // LANES


def _sc_body(pos_ci, pos_ei, neg_ci, neg_ei, case_t, ent_t,
             pos_out, neg_out,
             idx_pc, idx_pe, idx_nc, idx_ne,
             g_pc, g_pe, g_nc, g_ne,
             score_p, score_n,
             sem_pc, sem_pe, sem_nc, sem_ne):
  wid = lax.axis_index("s") * NUM_CORES + lax.axis_index("c")
  base = wid * BPW

  pltpu.sync_copy(pos_ci.at[pl.ds(base, BPW)], idx_pc)
  pltpu.sync_copy(pos_ei.at[pl.ds(base, BPW)], idx_pe)
  pltpu.sync_copy(neg_ci.at[pl.ds(base, BPW)], idx_nc)
  pltpu.sync_copy(neg_ei.at[pl.ds(base, BPW)], idx_ne)

  def fire(table, idx, dst, sem):
    return [
        pltpu.make_async_copy(table.at[c].at[idx], dst.at[c], sem)
        for c in range(DIM)
    ]

  cps_pc = fire(case_t, idx_pc, g_pc, sem_pc)
  cps_pe = fire(ent_t, idx_pe, g_pe, sem_pe)
  cps_nc = fire(case_t, idx_nc, g_nc, sem_nc)
  cps_ne = fire(ent_t, idx_ne, g_ne, sem_ne)
  for cps in (cps_pc, cps_pe, cps_nc, cps_ne):
    for cp in cps:
      cp.start()

  def drain(cps):
    for cp in cps:
      cp.wait()

  def dot_groups(rows_a, rows_b, score):
    def body(g, carry):
      off = g * LANES
      acc = rows_a[0, pl.ds(off, LANES)] * rows_b[0, pl.ds(off, LANES)]
      for c in range(1, DIM):
        acc = acc + rows_a[c, pl.ds(off, LANES)] * rows_b[c, pl.ds(off, LANES)]
      score[pl.ds(off, LANES)] = acc
      return carry
    lax.fori_loop(0, GROUPS, body, 0, unroll=False)

  drain(cps_pc)
  drain(cps_pe)
  dot_groups(g_pc, g_pe, score_p)
  pltpu.sync_copy(score_p, pos_out.at[pl.ds(base, BPW)])

  drain(cps_nc)
  drain(cps_ne)
  dot_groups(g_nc, g_ne, score_n)
  pltpu.sync_copy(score_n, neg_out.at[pl.ds(base, BPW)])


_sc_scores = functools.partial(
    pl.kernel,
    out_type=[
        jax.ShapeDtypeStruct((BATCH,), jnp.float32),
        jax.ShapeDtypeStruct((BATCH,), jnp.float32),
    ],
    mesh=plsc.VectorSubcoreMesh(
        core_axis_name="c", subcore_axis_name="s",
        num_cores=NUM_CORES, num_subcores=NUM_SUBCORES),
    compiler_params=pltpu.CompilerParams(use_tc_tiling_on_sc=False),
    scratch_types=[
        pltpu.VMEM((BPW,), jnp.int32),
        pltpu.VMEM((BPW,), jnp.int32),
        pltpu.VMEM((BPW,), jnp.int32),
        pltpu.VMEM((BPW,), jnp.int32),
        pltpu.VMEM((DIM, BPW), jnp.float32),
        pltpu.VMEM((DIM, BPW), jnp.float32),
        pltpu.VMEM((DIM, BPW), jnp.float32),
        pltpu.VMEM((DIM, BPW), jnp.float32),
        pltpu.VMEM((BPW,), jnp.float32),
        pltpu.VMEM((BPW,), jnp.float32),
        pltpu.SemaphoreType.DMA,
        pltpu.SemaphoreType.DMA,
        pltpu.SemaphoreType.DMA,
        pltpu.SemaphoreType.DMA,
    ],
)(_sc_body)


def _tc_loss_body(pos_ref, neg_ref, out_ref):
  ps = pos_ref[...]
  ns = neg_ref[...]

  def logsig(x):
    return jnp.minimum(x, 0.0) - jnp.log1p(jnp.exp(-jnp.abs(x)))

  total = jnp.sum(logsig(ps)) + jnp.sum(logsig(-ns))
  out_ref[0, 0] = -total


def kernel(pos_caseid, pos_entityid, neg_caseid, neg_entity,
           case_emb, entity_emb):
  pos_scores, neg_scores = _sc_scores(
      pos_caseid.astype(jnp.int32),
      pos_entityid.astype(jnp.int32),
      neg_caseid.astype(jnp.int32),
      neg_entity.astype(jnp.int32),
      case_emb.T, entity_emb.T)

  loss = pl.pallas_call(
      _tc_loss_body,
      out_shape=jax.ShapeDtypeStruct((1, 1), jnp.float32),
      out_specs=pl.BlockSpec(memory_space=pltpu.SMEM),
  )(pos_scores.reshape(128, 128), neg_scores.reshape(128, 128))
  return loss[0, 0]


# speedup vs baseline: 5.6585x; 5.6585x over previous
import functools

import jax
import jax.numpy as jnp
from jax import lax
from jax.experimental import pallas as pl
from jax.experimental.pallas import tpu as pltpu
from jax.experimental.pallas import tpu_sc as plsc

BATCH = 16384
DIM = 32
NUM_CORES = 2
NUM_SUBCORES = 16
LANES = 16
NUM_WORKERS = NUM_CORES * NUM_SUBCORES
BPW = BATCH // NUM_WORKERS
CHUNK = 128
NCHUNK = BPW // CHUNK
GROUPS = BPW // LANES


def _sc_body(pos_ci, pos_ei, neg_ci, neg_ei, case_emb, ent_emb,
             pos_out, neg_out,
             idx_pc, idx_pe, idx_nc, idx_ne,
             rows_pc, rows_pe, rows_nc, rows_ne,
             score_p, score_n, prod,
             sem_pc, sem_pe, sem_nc, sem_ne):
  wid = lax.axis_index("s") * NUM_CORES + lax.axis_index("c")
  base = wid * BPW

  for k in range(NCHUNK):
    sl = pl.ds(base + k * CHUNK, CHUNK)
    pltpu.sync_copy(pos_ci.at[sl], idx_pc.at[k])
    pltpu.sync_copy(pos_ei.at[sl], idx_pe.at[k])
    pltpu.sync_copy(neg_ci.at[sl], idx_nc.at[k])
    pltpu.sync_copy(neg_ei.at[sl], idx_ne.at[k])

  def fire(table, idx, rows, sem):
    return [
        pltpu.make_async_copy(table.at[idx.at[k]],
                              rows.at[pl.ds(k * CHUNK, CHUNK)], sem)
        for k in range(NCHUNK)
    ]

  cps_pc = fire(case_emb, idx_pc, rows_pc, sem_pc)
  cps_pe = fire(ent_emb, idx_pe, rows_pe, sem_pe)
  cps_nc = fire(case_emb, idx_nc, rows_nc, sem_nc)
  cps_ne = fire(ent_emb, idx_ne, rows_ne, sem_ne)
  for cps in (cps_pc, cps_pe, cps_nc, cps_ne):
    for cp in cps:
      cp.start()

  lane_iota = lax.iota(jnp.int32, LANES)

  def dot_groups(rows_a, rows_b, score):
    def body(g, carry):
      base_r = g * LANES
      for j in range(LANES):
        r = base_r + j
        a_lo = rows_a[r, pl.ds(0, LANES)]
        a_hi = rows_a[r, pl.ds(LANES, LANES)]
        b_lo = rows_b[r, pl.ds(0, LANES)]
        b_hi = rows_b[r, pl.ds(LANES, LANES)]
        p = a_lo * b_lo + a_hi * b_hi
        plsc.store_scatter(prod, [lane_iota * LANES + j], p)
      acc = prod[pl.ds(0, LANES)]
      for d in range(1, LANES):
        acc = acc + prod[pl.ds(d * LANES, LANES)]
      score[pl.ds(base_r, LANES)] = acc
      return carry
    lax.fori_loop(0, GROUPS, body, 0, unroll=False)

  for cp in cps_pc + cps_pe:
    cp.wait()
  dot_groups(rows_pc, rows_pe, score_p)
  pltpu.sync_copy(score_p, pos_out.at[pl.ds(base, BPW)])

  for cp in cps_nc + cps_ne:
    cp.wait()
  dot_groups(rows_nc, rows_ne, score_n)
  pltpu.sync_copy(score_n, neg_out.at[pl.ds(base, BPW)])


_sc_scores = functools.partial(
    pl.kernel,
    out_type=[
        jax.ShapeDtypeStruct((BATCH,), jnp.float32),
        jax.ShapeDtypeStruct((BATCH,), jnp.float32),
    ],
    mesh=plsc.VectorSubcoreMesh(
        core_axis_name="c", subcore_axis_name="s",
        num_cores=NUM_CORES, num_subcores=NUM_SUBCORES),
    compiler_params=pltpu.CompilerParams(
        needs_layout_passes=False, use_tc_tiling_on_sc=False),
    scratch_types=[
        pltpu.VMEM((NCHUNK, CHUNK), jnp.int32),
        pltpu.VMEM((NCHUNK, CHUNK), jnp.int32),
        pltpu.VMEM((NCHUNK, CHUNK), jnp.int32),
        pltpu.VMEM((NCHUNK, CHUNK), jnp.int32),
        pltpu.VMEM((BPW, DIM), jnp.float32),
        pltpu.VMEM((BPW, DIM), jnp.float32),
        pltpu.VMEM((BPW, DIM), jnp.float32),
        pltpu.VMEM((BPW, DIM), jnp.float32),
        pltpu.VMEM((BPW,), jnp.float32),
        pltpu.VMEM((BPW,), jnp.float32),
        pltpu.VMEM((LANES * LANES,), jnp.float32),
        pltpu.SemaphoreType.DMA,
        pltpu.SemaphoreType.DMA,
        pltpu.SemaphoreType.DMA,
        pltpu.SemaphoreType.DMA,
    ],
)(_sc_body)


def _tc_loss_body(pos_ref, neg_ref, out_ref):
  ps = pos_ref[...]
  ns = neg_ref[...]

  def logsig(x):
    return jnp.minimum(x, 0.0) - jnp.log1p(jnp.exp(-jnp.abs(x)))

  total = jnp.sum(logsig(ps)) + jnp.sum(logsig(-ns))
  out_ref[0, 0] = -total


def kernel(pos_caseid, pos_entityid, neg_caseid, neg_entity,
           case_emb, entity_emb):
  pos_scores, neg_scores = _sc_scores(
      pos_caseid.astype(jnp.int32),
      pos_entityid.astype(jnp.int32),
      neg_caseid.astype(jnp.int32),
      neg_entity.astype(jnp.int32),
      case_emb, entity_emb)

  loss = pl.pallas_call(
      _tc_loss_body,
      out_shape=jax.ShapeDtypeStruct((1, 1), jnp.float32),
      out_specs=pl.BlockSpec(memory_space=pltpu.SMEM),
  )(pos_scores.reshape(128, 128), neg_scores.reshape(128, 128))
  return loss[0, 0]


# speedup vs baseline: 7.3899x; 1.3060x over previous
import functools

import jax
import jax.numpy as jnp
from jax import lax
from jax.experimental import pallas as pl
from jax.experimental.pallas import tpu as pltpu
from jax.experimental.pallas import tpu_sc as plsc

BATCH = 16384
DIM = 32
NUM_CORES = 2
NUM_SUBCORES = 16
LANES = 16
NUM_WORKERS = NUM_CORES * NUM_SUBCORES
BPW = BATCH // NUM_WORKERS
CHUNK = 128
NCHUNK = BPW // CHUNK
CGROUPS = CHUNK // LANES

TABLE_ROWS = 1000000
PACK = 128 // DIM
RELAY_B = 512
RELAY_GRID = -(-TABLE_ROWS // (PACK * RELAY_B))
PACKED_ROWS = RELAY_GRID * RELAY_B


def _relayout_body(in_ref, out_ref):
  x = in_ref[...]
  out_ref[...] = x.reshape(128, RELAY_B).T


def _relayout(table_t):
  return pl.pallas_call(
      _relayout_body,
      grid=(RELAY_GRID,),
      in_specs=[pl.BlockSpec((DIM, PACK * RELAY_B), lambda i: (0, i))],
      out_specs=pl.BlockSpec((RELAY_B, 128), lambda i: (i, 0)),
      out_shape=jax.ShapeDtypeStruct((PACKED_ROWS, 128), jnp.float32),
  )(table_t)


def _sc_body(pos_ci, pos_ei, neg_ci, neg_ei, case_pk, ent_pk,
             pos_out, neg_out,
             idx_a, idx_b, pidx_a, pidx_b,
             rows_a0, rows_a1, rows_b0, rows_b1,
             score, prod,
             sem_a0, sem_a1, sem_b0, sem_b1):
  wid = lax.axis_index("s") * NUM_CORES + lax.axis_index("c")
  base = wid * BPW

  lane_iota = lax.iota(jnp.int32, LANES)

  def packed_row(v):
    return jnp.bitwise_or(
        jax.lax.shift_left(jax.lax.shift_right_logical(v, 11), 9),
        jnp.bitwise_and(v, 511))

  def stage_indices(src_a, src_b):
    for k in range(NCHUNK):
      sl = pl.ds(base + k * CHUNK, CHUNK)
      pltpu.sync_copy(src_a.at[sl], idx_a.at[k])
      pltpu.sync_copy(src_b.at[sl], idx_b.at[k])
    for k in range(NCHUNK):
      for g in range(CGROUPS):
        s = pl.ds(g * LANES, LANES)
        pidx_a[k, s] = packed_row(idx_a[k, s])
        pidx_b[k, s] = packed_row(idx_b[k, s])

  four_iota = lane_iota * PACK

  def chunk_dot(k, rows_ca, rows_eb):
    def body(g, carry):
      va = idx_a[k, pl.ds(g * LANES, LANES)]
      vb = idx_b[k, pl.ds(g * LANES, LANES)]
      for j in range(LANES):
        row = g * LANES + j
        row_v = jnp.full((LANES,), row, jnp.int32)
        pa = jnp.bitwise_and(jax.lax.shift_right_logical(va[j], 9), 3)
        pb = jnp.bitwise_and(jax.lax.shift_right_logical(vb[j], 9), 3)
        lane_a = four_iota + pa
        lane_b = four_iota + pb
        a_lo = plsc.load_gather(rows_ca, [row_v, lane_a])
        a_hi = plsc.load_gather(rows_ca, [row_v, lane_a + LANES * PACK])
        b_lo = plsc.load_gather(rows_eb, [row_v, lane_b])
        b_hi = plsc.load_gather(rows_eb, [row_v, lane_b + LANES * PACK])
        p = a_lo * b_lo + a_hi * b_hi
        plsc.store_scatter(prod, [lane_iota * LANES + j], p)
      acc = prod[pl.ds(0, LANES)]
      for d in range(1, LANES):
        acc = acc + prod[pl.ds(d * LANES, LANES)]
      score[pl.ds(k * CHUNK + g * LANES, LANES)] = acc
      return carry
    lax.fori_loop(0, CGROUPS, body, 0, unroll=False)

  def run_pair(src_a, src_b, out_hbm):
    stage_indices(src_a, src_b)
    bufs = [(rows_a0, rows_b0, sem_a0, sem_b0),
            (rows_a1, rows_b1, sem_a1, sem_b1)]

    def fire(k):
      ra, rb, sa, sb = bufs[k % 2]
      ca = pltpu.make_async_copy(case_pk.at[pidx_a.at[k]], ra, sa)
      cb = pltpu.make_async_copy(ent_pk.at[pidx_b.at[k]], rb, sb)
      ca.start(); cb.start()
      return ca, cb

    pend = fire(0)
    for k in range(NCHUNK):
      nxt = fire(k + 1) if k + 1 < NCHUNK else None
      ca, cb = pend
      ca.wait(); cb.wait()
      ra, rb, _, _ = bufs[k % 2]
      chunk_dot(k, ra, rb)
      pend = nxt
    pltpu.sync_copy(score, out_hbm.at[pl.ds(base, BPW)])

  run_pair(pos_ci, pos_ei, pos_out)
  run_pair(neg_ci, neg_ei, neg_out)


_sc_scores = functools.partial(
    pl.kernel,
    out_type=[
        jax.ShapeDtypeStruct((BATCH,), jnp.float32),
        jax.ShapeDtypeStruct((BATCH,), jnp.float32),
    ],
    mesh=plsc.VectorSubcoreMesh(
        core_axis_name="c", subcore_axis_name="s",
        num_cores=NUM_CORES, num_subcores=NUM_SUBCORES),
    compiler_params=pltpu.CompilerParams(
        needs_layout_passes=False, use_tc_tiling_on_sc=False),
    scratch_types=[
        pltpu.VMEM((NCHUNK, CHUNK), jnp.int32),
        pltpu.VMEM((NCHUNK, CHUNK), jnp.int32),
        pltpu.VMEM((NCHUNK, CHUNK), jnp.int32),
        pltpu.VMEM((NCHUNK, CHUNK), jnp.int32),
        pltpu.VMEM((CHUNK, 128), jnp.float32),
        pltpu.VMEM((CHUNK, 128), jnp.float32),
        pltpu.VMEM((CHUNK, 128), jnp.float32),
        pltpu.VMEM((CHUNK, 128), jnp.float32),
        pltpu.VMEM((BPW,), jnp.float32),
        pltpu.VMEM((LANES * LANES,), jnp.float32),
        pltpu.SemaphoreType.DMA,
        pltpu.SemaphoreType.DMA,
        pltpu.SemaphoreType.DMA,
        pltpu.SemaphoreType.DMA,
    ],
)(_sc_body)


def _tc_loss_body(pos_ref, neg_ref, out_ref):
  ps = pos_ref[...]
  ns = neg_ref[...]

  def logsig(x):
    return jnp.minimum(x, 0.0) - jnp.log1p(jnp.exp(-jnp.abs(x)))

  total = jnp.sum(logsig(ps)) + jnp.sum(logsig(-ns))
  out_ref[0, 0] = -total


def kernel(pos_caseid, pos_entityid, neg_caseid, neg_entity,
           case_emb, entity_emb):
  case_pk = _relayout(case_emb.T)
  ent_pk = _relayout(entity_emb.T)

  pos_scores, neg_scores = _sc_scores(
      pos_caseid.astype(jnp.int32),
      pos_entityid.astype(jnp.int32),
      neg_caseid.astype(jnp.int32),
      neg_entity.astype(jnp.int32),
      case_pk, ent_pk)

  loss = pl.pallas_call(
      _tc_loss_body,
      out_shape=jax.ShapeDtypeStruct((1, 1), jnp.float32),
      out_specs=pl.BlockSpec(memory_space=pltpu.SMEM),
  )(pos_scores.reshape(128, 128), neg_scores.reshape(128, 128))
  return loss[0, 0]
